# baseline (speedup 1.0000x reference)
import functools

import jax
import jax.numpy as jnp
from jax import lax
from jax.experimental import pallas as pl
from jax.experimental.pallas import tpu as pltpu
from jax.experimental.pallas import tpu_sc as plsc

HIDDEN = 64
NUM_WORKERS = 32
ROWS_PER_DMA = 1


def _emb_lookup_sc(x2d, emb):
    b, l = x2d.shape
    rpw = b // NUM_WORKERS
    nchunks = rpw // ROWS_PER_DMA
    chunk = ROWS_PER_DMA * l
    mesh = plsc.VectorSubcoreMesh(core_axis_name="c", subcore_axis_name="s")

    @functools.partial(
        pl.kernel,
        mesh=mesh,
        out_type=jax.ShapeDtypeStruct((b, l, HIDDEN), jnp.float32),
        compiler_params=pltpu.CompilerParams(use_tc_tiling_on_sc=False),
        scratch_types=[
            pltpu.VMEM((rpw, l), jnp.int32),
            pltpu.VMEM((chunk, HIDDEN), jnp.float32),
            pltpu.VMEM((chunk, HIDDEN), jnp.float32),
            pltpu.SemaphoreType.DMA,
            pltpu.SemaphoreType.DMA,
        ],
    )
    def k(idx_hbm, table_hbm, out_hbm, idx_v, buf0, buf1, gsem, wsem):
        wid = lax.axis_index("s") * 2 + lax.axis_index("c")
        base = wid * rpw
        pltpu.sync_copy(idx_hbm.at[pl.ds(base, rpw)], idx_v)

        bufs = (buf0, buf1)
        gathers = [None] * nchunks
        writes = [None] * nchunks

        def start_gather(g, buf):
            return pltpu.async_copy(table_hbm.at[idx_v.at[g]], buf, gsem)

        gathers[0] = start_gather(0, bufs[0])
        for g in range(nchunks):
            gathers[g].wait()
            if g >= 1:
                for w in writes[g - 1]:
                    w.wait()
            if g + 1 < nchunks:
                gathers[g + 1] = start_gather(g + 1, bufs[(g + 1) % 2])
            writes[g] = [
                pltpu.async_copy(bufs[g % 2], out_hbm.at[base + g], wsem)
            ]
        for w in writes[nchunks - 1]:
            w.wait()

    return k(x2d, emb)


def kernel(x, emb):
    if x.dtype != jnp.int32:
        x = x.astype(jnp.int32)
    return _emb_lookup_sc(x, emb)
